# baseline (speedup 1.0000x reference)
import functools

import jax
import jax.numpy as jnp
from jax import lax
from jax.experimental import pallas as pl
from jax.experimental.pallas import tpu as pltpu
from jax.experimental.pallas import tpu_sc as plsc

N = 32768
L = 16
NS = 16
CHUNK = N // NS
NV = CHUNK // L

_mesh = plsc.VectorSubcoreMesh(
    core_axis_name="c", subcore_axis_name="s", num_cores=1
)


@functools.partial(
    pl.kernel,
    mesh=_mesh,
    out_type=jax.ShapeDtypeStruct((N,), jnp.float32),
    scratch_types=[
        pltpu.VMEM((CHUNK,), jnp.float32),
        pltpu.VMEM((CHUNK,), jnp.float32),
        pltpu.VMEM((L,), jnp.float32),
        pltpu.VMEM((NS * L,), jnp.float32),
        pltpu.VMEM_SHARED((NS * L,), jnp.float32),
        pltpu.SemaphoreType.DMA,
        pltpu.SemaphoreType.DMA,
    ],
    compiler_params=pltpu.CompilerParams(needs_layout_passes=False),
)
def _sc_excl_cumsum(x_hbm, out_hbm, xv, ov, tv, allt, shared, sem1, sem2):
    sid = lax.axis_index("s")
    base = sid * CHUNK
    H = CHUNK // 2
    NVH = NV // 2

    in1 = pltpu.async_copy(x_hbm.at[pl.ds(base, H)], xv.at[pl.ds(0, H)], sem1)
    in2 = pltpu.async_copy(
        x_hbm.at[pl.ds(base + H, H)], xv.at[pl.ds(H, H)], sem2
    )
    in1.wait()

    @plsc.parallel_loop(0, NVH, unroll=8, carry=jnp.zeros((L,), jnp.float32))
    def acc1(i, a):
        return a + xv[pl.ds(i * L, L)]

    in2.wait()

    @plsc.parallel_loop(0, NVH, unroll=8, carry=acc1)
    def acc(i, a):
        return a + xv[pl.ds((NVH + i) * L, L)]

    total = jnp.sum(acc)

    tv[...] = jnp.full((L,), total, jnp.float32)
    pltpu.sync_copy(tv, shared.at[pl.ds(sid * L, L)])
    plsc.subcore_barrier()
    plsc.subcore_barrier()
    pltpu.sync_copy(shared, allt)

    lane = lax.iota(jnp.int32, L)
    t_vec = plsc.load_gather(allt, [lane * L])
    offset = jnp.sum(jnp.where(lane < sid, t_vec, jnp.zeros((L,), jnp.float32)))

    last = jnp.full((L,), L - 1, jnp.int32)

    @plsc.parallel_loop(
        0, NVH, unroll=8, carry=jnp.full((L,), offset, jnp.float32)
    )
    def carry_mid(i, carry):
        v = xv[pl.ds(i * L, L)]
        y = jnp.cumsum(v)
        ov[pl.ds(i * L, L)] = (y - v) + carry
        return carry + y.at[last].get(mode="promise_in_bounds")

    out1 = pltpu.async_copy(ov.at[pl.ds(0, H)], out_hbm.at[pl.ds(base, H)], sem1)

    @plsc.parallel_loop(0, NVH, unroll=8, carry=carry_mid)
    def _(i, carry):
        v = xv[pl.ds((NVH + i) * L, L)]
        y = jnp.cumsum(v)
        ov[pl.ds((NVH + i) * L, L)] = (y - v) + carry
        return carry + y.at[last].get(mode="promise_in_bounds")

    out2 = pltpu.async_copy(
        ov.at[pl.ds(H, H)], out_hbm.at[pl.ds(base + H, H)], sem2
    )
    out1.wait()
    out2.wait()


def kernel(input_0):
    return _sc_excl_cumsum(input_0)


# speedup vs baseline: 1.0113x; 1.0113x over previous
import functools

import jax
import jax.numpy as jnp
from jax import lax
from jax.experimental import pallas as pl
from jax.experimental.pallas import tpu as pltpu
from jax.experimental.pallas import tpu_sc as plsc

N = 32768
L = 16
NS = 16
CHUNK = N // NS
NV = CHUNK // L

_mesh = plsc.VectorSubcoreMesh(
    core_axis_name="c", subcore_axis_name="s", num_cores=1
)


@functools.partial(
    pl.kernel,
    mesh=_mesh,
    out_type=jax.ShapeDtypeStruct((N,), jnp.float32),
    scratch_types=[
        pltpu.VMEM((CHUNK,), jnp.float32),
        pltpu.VMEM((CHUNK,), jnp.float32),
        pltpu.VMEM((L,), jnp.float32),
        pltpu.VMEM((NS * L,), jnp.float32),
        pltpu.VMEM_SHARED((NS * L,), jnp.float32),
    ],
    compiler_params=pltpu.CompilerParams(needs_layout_passes=False),
)
def _sc_excl_cumsum(x_hbm, out_hbm, xv, ov, tv, allt, shared):
    sid = lax.axis_index("s")
    base = sid * CHUNK

    pltpu.sync_copy(x_hbm.at[pl.ds(base, CHUNK)], xv)

    @plsc.parallel_loop(0, NV, unroll=8, carry=jnp.zeros((L,), jnp.float32))
    def acc(i, a):
        return a + xv[pl.ds(i * L, L)]

    total = jnp.sum(acc)

    tv[...] = jnp.full((L,), total, jnp.float32)
    pltpu.sync_copy(tv, shared.at[pl.ds(sid * L, L)])
    plsc.subcore_barrier()
    plsc.subcore_barrier()
    pltpu.sync_copy(shared, allt)

    lane = lax.iota(jnp.int32, L)
    t_vec = plsc.load_gather(allt, [lane * L])
    offset = jnp.sum(jnp.where(lane < sid, t_vec, jnp.zeros((L,), jnp.float32)))

    last = jnp.full((L,), L - 1, jnp.int32)

    @plsc.parallel_loop(
        0, NV, unroll=8, carry=jnp.full((L,), offset, jnp.float32)
    )
    def _(i, carry):
        v = xv[pl.ds(i * L, L)]
        y = jnp.cumsum(v)
        ov[pl.ds(i * L, L)] = (y - v) + carry
        return carry + y.at[last].get(mode="promise_in_bounds")

    pltpu.sync_copy(ov, out_hbm.at[pl.ds(base, CHUNK)])


def kernel(input_0):
    return _sc_excl_cumsum(input_0)
